# baseline (speedup 1.0000x reference)
import functools

import jax
import jax.numpy as jnp
from jax import lax
from jax.experimental import pallas as pl
from jax.experimental.pallas import tpu as pltpu
from jax.experimental.pallas import tpu_sc as plsc

B = 16384
EMB = 32
NFEAT = 26
NC = 2
NS = 16
NW = NC * NS
BPW = B // NW


@functools.partial(
    pl.kernel,
    mesh=plsc.VectorSubcoreMesh(core_axis_name="c", subcore_axis_name="s"),
    out_type=tuple(
        jax.ShapeDtypeStruct((B, EMB), jnp.float32) for _ in range(NFEAT)
    ),
    scratch_types=[
        pltpu.VMEM((NFEAT, BPW), jnp.int32),
        pltpu.VMEM((BPW, EMB), jnp.float32),
        pltpu.SemaphoreType.DMA,
    ],
    compiler_params=pltpu.CompilerParams(use_tc_tiling_on_sc=False),
)
def _lookup(*refs):
    idx_hbm = refs[0]
    tables = refs[1:1 + NFEAT]
    outs = refs[1 + NFEAT:1 + 2 * NFEAT]
    idx_v, buf_v, sem = refs[1 + 2 * NFEAT:]

    wid = lax.axis_index("s") * NC + lax.axis_index("c")
    base = wid * BPW
    pltpu.sync_copy(idx_hbm.at[wid], idx_v)
    for f in range(NFEAT):
        pltpu.async_copy(tables[f].at[idx_v.at[f]], buf_v, sem).wait()
        pltpu.sync_copy(buf_v, outs[f].at[pl.ds(base, BPW), :])


def kernel(f00, W_f00, f01, W_f01, f02, W_f02, f03, W_f03, f04, W_f04,
           f05, W_f05, f06, W_f06, f07, W_f07, f08, W_f08, f09, W_f09,
           f10, W_f10, f11, W_f11, f12, W_f12, f13, W_f13, f14, W_f14,
           f15, W_f15, f16, W_f16, f17, W_f17, f18, W_f18, f19, W_f19,
           f20, W_f20, f21, W_f21, f22, W_f22, f23, W_f23, f24, W_f24,
           f25, W_f25):
    idxs = [f00, f01, f02, f03, f04, f05, f06, f07, f08, f09, f10, f11,
            f12, f13, f14, f15, f16, f17, f18, f19, f20, f21, f22, f23,
            f24, f25]
    tables = [W_f00, W_f01, W_f02, W_f03, W_f04, W_f05, W_f06, W_f07,
              W_f08, W_f09, W_f10, W_f11, W_f12, W_f13, W_f14, W_f15,
              W_f16, W_f17, W_f18, W_f19, W_f20, W_f21, W_f22, W_f23,
              W_f24, W_f25]
    idx_all = jnp.stack(idxs).reshape(NFEAT, NW, BPW).transpose(1, 0, 2)
    embs = _lookup(idx_all, *tables)
    return jnp.concatenate(embs, axis=-1)


# speedup vs baseline: 1.1958x; 1.1958x over previous
import functools

import jax
import jax.numpy as jnp
from jax import lax
from jax.experimental import pallas as pl
from jax.experimental.pallas import tpu as pltpu
from jax.experimental.pallas import tpu_sc as plsc

B = 16384
EMB = 32
NFEAT = 26
OUTW = NFEAT * EMB
NC = 2
NS = 16
NW = NC * NS
BPW = B // NW
CH = 128
NCH = BPW // CH


@functools.partial(
    pl.kernel,
    mesh=plsc.VectorSubcoreMesh(core_axis_name="c", subcore_axis_name="s"),
    out_type=jax.ShapeDtypeStruct((B, OUTW), jnp.float32),
    scratch_types=[
        pltpu.VMEM((NFEAT, BPW), jnp.int32),
        pltpu.VMEM((2, BPW, EMB), jnp.float32),
        pltpu.SemaphoreType.DMA,
        pltpu.SemaphoreType.DMA,
    ],
    compiler_params=pltpu.CompilerParams(use_tc_tiling_on_sc=False),
)
def _lookup_concat(*refs):
    idx_hbm = refs[0]
    tables = refs[1:1 + NFEAT]
    out_hbm = refs[1 + NFEAT]
    idx_v, buf_v, gsem, wsem = refs[2 + NFEAT:]

    wid = lax.axis_index("s") * NC + lax.axis_index("c")
    base = wid * BPW
    pltpu.sync_copy(idx_hbm.at[wid], idx_v)

    gathers = [
        pltpu.make_async_copy(
            tables[f].at[idx_v.at[f]], buf_v.at[f % 2], gsem
        )
        for f in range(NFEAT)
    ]
    writes = [
        pltpu.make_async_copy(
            buf_v.at[f % 2],
            out_hbm.at[pl.ds(base, BPW), pl.ds(f * EMB, EMB)],
            wsem,
        )
        for f in range(NFEAT)
    ]
    gathers[0].start()
    for f in range(NFEAT):
        if f + 1 < NFEAT:
            if f >= 1:
                writes[f - 1].wait()
            gathers[f + 1].start()
        gathers[f].wait()
        writes[f].start()
    writes[NFEAT - 2].wait()
    writes[NFEAT - 1].wait()


def kernel(f00, W_f00, f01, W_f01, f02, W_f02, f03, W_f03, f04, W_f04,
           f05, W_f05, f06, W_f06, f07, W_f07, f08, W_f08, f09, W_f09,
           f10, W_f10, f11, W_f11, f12, W_f12, f13, W_f13, f14, W_f14,
           f15, W_f15, f16, W_f16, f17, W_f17, f18, W_f18, f19, W_f19,
           f20, W_f20, f21, W_f21, f22, W_f22, f23, W_f23, f24, W_f24,
           f25, W_f25):
    idxs = [f00, f01, f02, f03, f04, f05, f06, f07, f08, f09, f10, f11,
            f12, f13, f14, f15, f16, f17, f18, f19, f20, f21, f22, f23,
            f24, f25]
    tables = [W_f00, W_f01, W_f02, W_f03, W_f04, W_f05, W_f06, W_f07,
              W_f08, W_f09, W_f10, W_f11, W_f12, W_f13, W_f14, W_f15,
              W_f16, W_f17, W_f18, W_f19, W_f20, W_f21, W_f22, W_f23,
              W_f24, W_f25]
    idx_all = jnp.stack(idxs).reshape(NFEAT, NW, BPW).transpose(1, 0, 2)
    return _lookup_concat(idx_all, *tables)
